# Initial kernel scaffold; baseline (speedup 1.0000x reference)
import jax
import jax.numpy as jnp
from jax.experimental import pallas as pl


def kernel(x, W):
    raise NotImplementedError("write your pallas kernel here")


# speedup vs baseline: 1.0104x; 1.0104x over previous
import functools

import jax
import jax.numpy as jnp
from jax import lax
from jax.experimental import pallas as pl
from jax.experimental.pallas import tpu as pltpu
from jax.experimental.pallas import tpu_sc as plsc

_F = 26
_V = 100000
_D = 32
_B = 4096
_NC = 2
_NS = 16
_NW = _NC * _NS
_L = 16
_CHUNK = 128
_ROWS_W = _B * _F // _NW
_NCHUNK = _ROWS_W // _CHUNK

_mesh = plsc.VectorSubcoreMesh(core_axis_name="c", subcore_axis_name="s")


@functools.partial(
    pl.kernel,
    mesh=_mesh,
    out_type=jax.ShapeDtypeStruct((_B * _F, _D), jnp.float32),
    scratch_types=[
        pltpu.VMEM((_NCHUNK, _CHUNK), jnp.int32),
        pltpu.VMEM((_NCHUNK, _CHUNK), jnp.int32),
        pltpu.VMEM((_ROWS_W, _D), jnp.float32),
        pltpu.SemaphoreType.DMA,
    ],
    compiler_params=pltpu.CompilerParams(use_tc_tiling_on_sc=False),
)
def _emb_lookup(table, idx3d, offs, out, idx_v, off_v, rows_v, sem):
    wid = lax.axis_index("s") * _NC + lax.axis_index("c")
    pltpu.sync_copy(idx3d.at[wid], idx_v)
    pltpu.sync_copy(offs, off_v)

    def _add_row(j, _):
        def _add_strip(k, _):
            s = pl.ds(k * _L, _L)
            idx_v[j, s] = idx_v[j, s] + off_v[j, s]
            return _
        return lax.fori_loop(0, _CHUNK // _L, _add_strip, _)

    lax.fori_loop(0, _NCHUNK, _add_row, None)

    def _fire(j, _):
        pltpu.async_copy(
            table.at[idx_v.at[j]], rows_v.at[pl.ds(j * _CHUNK, _CHUNK)], sem)
        return _

    lax.fori_loop(0, _NCHUNK, _fire, None)

    def _drain(j, _):
        pltpu.make_async_copy(
            table.at[idx_v.at[j]], rows_v.at[pl.ds(j * _CHUNK, _CHUNK)], sem
        ).wait()
        return _

    lax.fori_loop(0, _NCHUNK, _drain, None)

    pltpu.sync_copy(rows_v, out.at[pl.ds(wid * _ROWS_W, _ROWS_W)])


def kernel(x, W):
    table = W.reshape(_F * _V, _D)
    idx = x.astype(jnp.int32).reshape(_NW, _NCHUNK, _CHUNK)
    offs = ((jnp.arange(_ROWS_W, dtype=jnp.int32) % _F) * _V).reshape(
        _NCHUNK, _CHUNK)
    out = _emb_lookup(table, idx, offs)
    return out.reshape(_B, _F, _D)


# speedup vs baseline: 1.3911x; 1.3768x over previous
import functools

import jax
import jax.numpy as jnp
from jax import lax
from jax.experimental import pallas as pl
from jax.experimental.pallas import tpu as pltpu
from jax.experimental.pallas import tpu_sc as plsc

_F = 26
_V = 100000
_D = 32
_B = 4096
_NC = 2
_NS = 16
_NW = _NC * _NS
_L = 16
_CHUNK = 128
_ROWS_W = _B * _F // _NW
_NCHUNK = _ROWS_W // _CHUNK

_mesh = plsc.VectorSubcoreMesh(core_axis_name="c", subcore_axis_name="s")


@functools.partial(
    pl.kernel,
    mesh=_mesh,
    out_type=jax.ShapeDtypeStruct((_B * _F, _D), jnp.float32),
    scratch_types=[
        pltpu.VMEM((_NCHUNK, _CHUNK), jnp.int32),
        pltpu.VMEM((_NCHUNK, _CHUNK), jnp.int32),
        pltpu.VMEM((_CHUNK, _D), jnp.float32),
        pltpu.SemaphoreType.DMA,
    ],
)
def _gather_direct(table, idx3d, fld, out, idx_v, fld_v, row_v, sem):
    wid = lax.axis_index("s") * _NC + lax.axis_index("c")
    pltpu.sync_copy(idx3d.at[wid], idx_v)
    pltpu.sync_copy(fld, fld_v)

    def _chunk(j, _):
        def _strip(k, _):
            s = pl.ds(k * _L, _L)
            rv = idx_v[j, s]
            fv = fld_v[j, s]
            for i in range(_L):
                pltpu.async_copy(
                    table.at[fv[i], pl.ds(rv[i], 1), :],
                    row_v.at[pl.ds(k * _L + i, 1), :],
                    sem,
                )
            return _

        lax.fori_loop(0, _CHUNK // _L, _strip, None)
        out_slc = out.at[pl.ds(wid * _ROWS_W + j * _CHUNK, _CHUNK)]
        pltpu.make_async_copy(out_slc, row_v, sem).wait()
        pltpu.sync_copy(row_v, out_slc)
        return _

    lax.fori_loop(0, _NCHUNK, _chunk, None)


def kernel(x, W):
    idx = x.astype(jnp.int32).reshape(_NW, _NCHUNK, _CHUNK)
    fld = (jnp.arange(_ROWS_W, dtype=jnp.int32) % _F).reshape(_NCHUNK, _CHUNK)
    out = _gather_direct(W, idx, fld)
    return out.reshape(_B, _F, _D)


# speedup vs baseline: 2.0140x; 1.4477x over previous
import functools

import jax
import jax.numpy as jnp
from jax import lax
from jax.experimental import pallas as pl
from jax.experimental.pallas import tpu as pltpu
from jax.experimental.pallas import tpu_sc as plsc

_F = 26
_V = 100000
_D = 32
_B = 4096
_NC = 2
_NS = 16
_NW = _NC * _NS
_L = 16
_CHUNK = 128
_POS_W = _B * _F // _NW
_NCHUNK = _POS_W // _CHUNK

_mesh = plsc.VectorSubcoreMesh(core_axis_name="c", subcore_axis_name="s")


@functools.partial(
    pl.kernel,
    mesh=_mesh,
    out_type=jax.ShapeDtypeStruct((_F * _D, _B), jnp.float32),
    scratch_types=[
        pltpu.VMEM((_NCHUNK, _CHUNK), jnp.int32),
        pltpu.VMEM((_D, _POS_W), jnp.float32),
        pltpu.SemaphoreType.DMA,
    ],
    compiler_params=pltpu.CompilerParams(use_tc_tiling_on_sc=False),
)
def _emb_t(tableT, idxT3, out, idx_v, val_v, sem):
    wid = lax.axis_index("s") * _NC + lax.axis_index("c")
    base = wid * _POS_W
    pltpu.sync_copy(idxT3.at[wid], idx_v)

    def _fire(j, _):
        f = (base + j * _CHUNK) // _B
        for d in range(_D):
            pltpu.async_copy(
                tableT.at[f * _D + d].at[idx_v.at[j]],
                val_v.at[d, pl.ds(j * _CHUNK, _CHUNK)],
                sem,
            )
        return _

    lax.fori_loop(0, _NCHUNK, _fire, None)

    pltpu.make_async_copy(
        out.at[pl.ds(0, _D), pl.ds(0, _POS_W)], val_v, sem).wait()

    def _wb(j, _):
        p = base + j * _CHUNK
        f = p // _B
        b = p - f * _B
        pltpu.sync_copy(
            val_v.at[:, pl.ds(j * _CHUNK, _CHUNK)],
            out.at[pl.ds(f * _D, _D), pl.ds(b, _CHUNK)],
        )
        return _

    lax.fori_loop(0, _NCHUNK, _wb, None)


def kernel(x, W):
    tableT = W.transpose(0, 2, 1).reshape(_F * _D, _V)
    idxT = x.astype(jnp.int32).T.reshape(_NW, _NCHUNK, _CHUNK)
    outT = _emb_t(tableT, idxT)
    return outT.reshape(_F, _D, _B).transpose(2, 0, 1)
